# baseline (speedup 1.0000x reference)
import jax
import jax.numpy as jnp
from jax.experimental import pallas as pl

_ROWS = 10000
_COLS = 256
_BLOCK_ROWS = 2000


def _copy_body(x_ref, e_ref, xo_ref, eo_ref):
    xo_ref[...] = x_ref[...]
    eo_ref[...] = e_ref[...]


def kernel(x, edge_index, edge_attr):
    del edge_index
    e2 = edge_attr.reshape(_ROWS, _COLS)
    spec = pl.BlockSpec((_BLOCK_ROWS, _COLS), lambda i: (i, 0))
    x_out, e_out = pl.pallas_call(
        _copy_body,
        grid=(_ROWS // _BLOCK_ROWS,),
        in_specs=[spec, spec],
        out_specs=[spec, spec],
        out_shape=[
            jax.ShapeDtypeStruct((_ROWS, _COLS), x.dtype),
            jax.ShapeDtypeStruct((_ROWS, _COLS), edge_attr.dtype),
        ],
    )(x, e2)
    return (x_out, e_out.reshape(edge_attr.shape))
